# baseline (speedup 1.0000x reference)
import functools

import jax
import jax.numpy as jnp
from jax import lax
from jax.experimental import pallas as pl
from jax.experimental.pallas import tpu as pltpu
from jax.experimental.pallas import tpu_sc as plsc

NC = 2
NS = 16
NW = NC * NS
GMAX = 128
CS = 2
NBUF = 4


def _gather_slices(rows):
    out, off = [], 0
    while off < rows:
        n = min(GMAX, rows - off)
        out.append((off, n))
        off += n
    return out


def _emb_body(S, D, n_per_w, tok_hbm, e_hbm, p_hbm, out_hbm,
              idx_v, rows_v, sem_in, sem_g, sem_w):
    wid = lax.axis_index("s") * NC + lax.axis_index("c")
    base = wid * n_per_w
    cr = CS * S
    n_chunks = n_per_w // cr
    slices = _gather_slices(cr)

    def outer(g, carry):
        descs_in = []
        for b in range(NBUF):
            chunk = g + b

            @pl.when(chunk >= NBUF)
            def _(b=b):
                pltpu.make_async_copy(rows_v[b],
                                      out_hbm.at[pl.ds(0, cr)],
                                      sem_w[b]).wait()

            row0 = base + chunk * cr
            d1 = pltpu.async_copy(tok_hbm.at[pl.ds(row0, cr)], idx_v[b],
                                  sem_in[b])
            d2 = pltpu.async_copy(p_hbm, rows_v[b], sem_in[b])
            descs_in.append((d1, d2))
        gdescs = []
        for b in range(NBUF):
            d1, d2 = descs_in[b]
            d1.wait()
            d2.wait()
            gdescs.append([
                pltpu.async_copy(e_hbm.at[idx_v[b].at[pl.ds(off, n)]],
                                 rows_v[b].at[pl.ds(off, n)], sem_g[b],
                                 add=True)
                for off, n in slices
            ])
        for b in range(NBUF):
            for d in gdescs[b]:
                d.wait()
            row0 = base + (g + b) * cr
            pltpu.async_copy(rows_v[b], out_hbm.at[pl.ds(row0, cr)], sem_w[b])
        return carry

    lax.fori_loop(0, n_chunks // NBUF, lambda i, c: outer(i * NBUF, c), 0)
    for b in range(NBUF):
        pltpu.make_async_copy(rows_v[b], out_hbm.at[pl.ds(0, cr)],
                              sem_w[b]).wait()


def _mask_body(tok_ref, out_ref):
    out_ref[...] = tok_ref[...] != 0


def _transpose_body(*refs):
    bands, out_ref = refs[:-1], refs[-1]
    out_ref[...] = jnp.concatenate([r[...] for r in bands], axis=0).T


def _row_major_table(E):
    V, D = E.shape
    CB = 8192
    BAND = 16
    grid = (V + CB - 1) // CB
    return pl.pallas_call(
        _transpose_body,
        grid=(grid,),
        in_specs=[
            pl.BlockSpec((BAND, CB), functools.partial(lambda k, i: (k, i), k))
            for k in range(D // BAND)
        ],
        out_specs=pl.BlockSpec((CB, D), lambda i: (i, 0)),
        out_shape=jax.ShapeDtypeStruct((V, D), jnp.float32),
    )(*([E.T] * (D // BAND)))


def kernel(token_batch, E, P):
    B, S = token_batch.shape
    V, D = E.shape
    N = B * S
    n_per_w = N // NW
    tok_flat = token_batch.reshape(N)
    p_tiled = jnp.tile(P, (CS, 1))

    cr = CS * S
    mesh = plsc.VectorSubcoreMesh(core_axis_name="c", subcore_axis_name="s")
    emb = functools.partial(
        pl.kernel,
        mesh=mesh,
        out_type=jax.ShapeDtypeStruct((N, D), jnp.float32),
        scratch_types=[
            [pltpu.VMEM((cr,), jnp.int32) for _ in range(NBUF)],
            [pltpu.VMEM((cr, D), jnp.float32) for _ in range(NBUF)],
            [pltpu.SemaphoreType.DMA for _ in range(NBUF)],
            [pltpu.SemaphoreType.DMA for _ in range(NBUF)],
            [pltpu.SemaphoreType.DMA for _ in range(NBUF)],
        ],
        compiler_params=pltpu.CompilerParams(use_tc_tiling_on_sc=False),
    )(functools.partial(_emb_body, S, D, n_per_w))

    x_flat = emb(tok_flat, _row_major_table(E), p_tiled)

    mask = pl.pallas_call(
        _mask_body,
        out_shape=jax.ShapeDtypeStruct((B, S), jnp.bool_),
    )(token_batch)

    return (x_flat.reshape(B, S, D), mask)


# speedup vs baseline: 1.0628x; 1.0628x over previous
import functools

import jax
import jax.numpy as jnp
from jax import lax
from jax.experimental import pallas as pl
from jax.experimental.pallas import tpu as pltpu
from jax.experimental.pallas import tpu_sc as plsc

NC = 2
NS = 16
NW = NC * NS
GMAX = 128
CS = 2
NBUF = 4


def _gather_slices(rows):
    out, off = [], 0
    while off < rows:
        n = min(GMAX, rows - off)
        out.append((off, n))
        off += n
    return out


def _emb_body(S, D, n_per_w, tok_hbm, e_hbm, p_hbm, out_hbm,
              idx_v, rows_v, sem_in, sem_g, sem_w):
    wid = lax.axis_index("s") * NC + lax.axis_index("c")
    base = wid * n_per_w
    cr = CS * S
    n_chunks = n_per_w // cr
    slices = _gather_slices(cr)

    def outer(g, carry):
        descs_in = []
        for b in range(NBUF):
            chunk = g + b

            @pl.when(chunk >= NBUF)
            def _(b=b):
                pltpu.make_async_copy(rows_v[b],
                                      out_hbm.at[pl.ds(0, cr)],
                                      sem_w[b]).wait()

            row0 = base + chunk * cr
            d1 = pltpu.async_copy(tok_hbm.at[pl.ds(row0, cr)], idx_v[b],
                                  sem_in[b])
            d2 = pltpu.async_copy(p_hbm, rows_v[b], sem_in[b])
            descs_in.append((d1, d2))
        gdescs = []
        for b in range(NBUF):
            d1, d2 = descs_in[b]
            d1.wait()
            d2.wait()
            gdescs.append([
                pltpu.async_copy(e_hbm.at[idx_v[b].at[pl.ds(off, n)]],
                                 rows_v[b].at[pl.ds(off, n)], sem_g[b],
                                 add=True)
                for off, n in slices
            ])
        for b in range(NBUF):
            for d in gdescs[b]:
                d.wait()
            row0 = base + (g + b) * cr
            pltpu.async_copy(rows_v[b], out_hbm.at[pl.ds(row0, cr)], sem_w[b])
        return carry

    lax.fori_loop(0, n_chunks // NBUF, lambda i, c: outer(i * NBUF, c), 0)
    for b in range(NBUF):
        pltpu.make_async_copy(rows_v[b], out_hbm.at[pl.ds(0, cr)],
                              sem_w[b]).wait()


def _mask_body(tok_ref, out_ref):
    out_ref[...] = tok_ref[...] != 0


def kernel(token_batch, E, P):
    B, S = token_batch.shape
    V, D = E.shape
    N = B * S
    n_per_w = N // NW
    tok_flat = token_batch.reshape(N)
    p_tiled = jnp.tile(P, (CS, 1))

    cr = CS * S
    mesh = plsc.VectorSubcoreMesh(core_axis_name="c", subcore_axis_name="s")
    emb = functools.partial(
        pl.kernel,
        mesh=mesh,
        out_type=jax.ShapeDtypeStruct((N, D), jnp.float32),
        scratch_types=[
            [pltpu.VMEM((cr,), jnp.int32) for _ in range(NBUF)],
            [pltpu.VMEM((cr, D), jnp.float32) for _ in range(NBUF)],
            [pltpu.SemaphoreType.DMA for _ in range(NBUF)],
            [pltpu.SemaphoreType.DMA for _ in range(NBUF)],
            [pltpu.SemaphoreType.DMA for _ in range(NBUF)],
        ],
        compiler_params=pltpu.CompilerParams(use_tc_tiling_on_sc=False),
    )(functools.partial(_emb_body, S, D, n_per_w))

    x_flat = emb(tok_flat, E, p_tiled)

    mask = pl.pallas_call(
        _mask_body,
        out_shape=jax.ShapeDtypeStruct((B, S), jnp.bool_),
    )(token_batch)

    return (x_flat.reshape(B, S, D), mask)
